# baseline (speedup 1.0000x reference)
import functools

import jax
import jax.numpy as jnp
from jax import lax
from jax.experimental import pallas as pl
from jax.experimental.pallas import tpu as pltpu
from jax.experimental.pallas import tpu_sc as plsc

V0 = 1000000
V1 = 100000
D = 64
B = 16384
L = 50
NC = 2
NS = 16
LANES = 16
NW = NC * NS

K = 128
NB = 4
DT = D // 8
NCHUNK = L * (B // K)
CPW = NCHUNK // NW
NJO = CPW // NB
OUT_FLAT = L * D * B


@functools.lru_cache(maxsize=None)
def _build():
    mesh = plsc.VectorSubcoreMesh(core_axis_name="c", subcore_axis_name="s")

    @functools.partial(
        pl.kernel,
        out_type=jax.ShapeDtypeStruct((OUT_FLAT,), jnp.float32),
        mesh=mesh,
        compiler_params=pltpu.CompilerParams(
            use_tc_tiling_on_sc=False, needs_layout_passes=False),
        scratch_types=[
            [pltpu.VMEM((K,), jnp.int32)] * NB,
            [pltpu.VMEM((K,), jnp.int32)] * NB,
            [pltpu.VMEM((K,), jnp.int32)] * NB,
            [pltpu.VMEM((K,), jnp.int32)] * NB,
            [pltpu.VMEM((2 * K, D), jnp.float32)] * NB,
            [pltpu.VMEM((D * K,), jnp.float32)] * NB,
            [pltpu.SemaphoreType.DMA] * NB,
            [pltpu.SemaphoreType.DMA] * NB,
            [pltpu.SemaphoreType.DMA] * NB,
            [pltpu.SemaphoreType.DMA] * NB,
        ],
    )
    def emb(t0, t1, xtf, out, idxv, l0, l1, msk, rows01, tbuf,
            semi, semg0, semg1, semw):
        wid = lax.axis_index("s") * NC + lax.axis_index("c")
        c0 = wid * CPW
        iota = lax.broadcasted_iota(jnp.int32, (LANES,), 0)

        def xoff(c):
            return (c // (B // K)) * B + (c % (B // K)) * K

        def obase(c):
            l = c // (B // K)
            vb = c % (B // K)
            return (l * 8 * (B // K) + vb) * (8 * K)

        def owrite(pb, c, fire):
            ob = obase(c)
            for dt in range(DT):
                src = tbuf[pb].at[pl.ds(dt * 8 * K, 8 * K)]
                dst = out.at[pl.ds(ob + dt * (B // K) * 8 * K, 8 * K)]
                if fire:
                    pltpu.async_copy(src, dst, semw[pb])
                else:
                    pltpu.make_async_copy(src, dst, semw[pb]).wait()

        def process(pb, c):
            pltpu.make_async_copy(
                t0.at[l0[pb]], rows01[pb].at[pl.ds(0, K)], semg0[pb]).wait()
            pltpu.make_async_copy(
                t1.at[l1[pb]], rows01[pb].at[pl.ds(K, K)], semg1[pb]).wait()

            @pl.when(c - c0 >= NB)
            def _():
                owrite(pb, c - NB, fire=False)

            def col(g, _):
                rowv = (g * LANES + iota
                        + msk[pb][pl.ds(g * LANES, LANES)] * K)
                for d in range(D):
                    dvec = jnp.full((LANES,), d, jnp.int32)
                    val = plsc.load_gather(rows01[pb], [rowv, dvec])
                    tbuf[pb][pl.ds(d * K + g * LANES, LANES)] = val
                return 0

            lax.fori_loop(0, K // LANES, col, 0)
            owrite(pb, c, fire=True)

        for b in range(NB):
            pltpu.async_copy(
                xtf.at[pl.ds(xoff(c0 + b), K)], idxv[b], semi[b])

        def outer(jo, _):
            for b in range(NB):
                j = jo * NB + b
                c = c0 + j
                pltpu.make_async_copy(
                    xtf.at[pl.ds(xoff(c), K)], idxv[b], semi[b]).wait()

                def vec(v, _, _b=b):
                    sl = pl.ds(v * LANES, LANES)
                    vi = idxv[_b][sl]
                    m1 = vi >= V0
                    l0[_b][sl] = jnp.minimum(
                        jnp.where(m1, vi - V0, vi), V0 - 1)
                    l1[_b][sl] = jnp.where(
                        m1, jnp.minimum(vi - V0, V1 - 1), vi & 0xFFFF)
                    msk[_b][sl] = m1.astype(jnp.int32)
                    return 0

                lax.fori_loop(0, K // LANES, vec, 0)

                @pl.when(jo < NJO - 1)
                def _():
                    pltpu.async_copy(
                        xtf.at[pl.ds(xoff(c + NB), K)], idxv[b], semi[b])

                pltpu.async_copy(
                    t0.at[l0[b]], rows01[b].at[pl.ds(0, K)], semg0[b])
                pltpu.async_copy(
                    t1.at[l1[b]], rows01[b].at[pl.ds(K, K)], semg1[b])

                @pl.when(j > 0)
                def _(_pb=(b - 1) % NB, _c=c - 1):
                    process(_pb, _c)
            return 0

        lax.fori_loop(0, NJO, outer, 0)

        process(NB - 1, c0 + CPW - 1)
        for b in range(NB):
            owrite(b, c0 + CPW - NB + b, fire=False)

    return emb


@jax.jit
def kernel(table0, table1, x):
    xtf = x.T.reshape(B * L)
    flat = _build()(table0, table1, xtf)
    out5 = flat.reshape(L, D // 8, B // K, 8, K)
    return out5.transpose(2, 4, 0, 1, 3).reshape(B, L, D)


# speedup vs baseline: 1.5539x; 1.5539x over previous
import functools

import numpy as _np

import jax
import jax.numpy as jnp
from jax import lax
from jax.experimental import pallas as pl
from jax.experimental.pallas import tpu as pltpu
from jax.experimental.pallas import tpu_sc as plsc

V0 = 1000000
V1 = 100000
D = 64
B = 16384
L = 50
NC = 2
NS = 16
LANES = 16
NW = NC * NS

K = 128
NB = 4
DT = D // 8
NCHUNK = L * (B // K)
CPW = NCHUNK // NW
NJO = CPW // NB
OUT_FLAT = L * D * B


@functools.lru_cache(maxsize=None)
def _build():
    mesh = plsc.VectorSubcoreMesh(core_axis_name="c", subcore_axis_name="s")

    @functools.partial(
        pl.kernel,
        out_type=jax.ShapeDtypeStruct((OUT_FLAT,), jnp.float32),
        mesh=mesh,
        compiler_params=pltpu.CompilerParams(
            use_tc_tiling_on_sc=False, needs_layout_passes=False),
        scratch_types=[
            [pltpu.VMEM((K,), jnp.int32)] * NB,
            [pltpu.VMEM((K,), jnp.int32)] * NB,
            [pltpu.VMEM((K,), jnp.int32)] * NB,
            [pltpu.VMEM((K,), jnp.int32)] * NB,
            [pltpu.VMEM((2 * K, D), jnp.float32)] * NB,
            [pltpu.VMEM((D * K,), jnp.float32)] * NB,
            [pltpu.SemaphoreType.DMA] * NB,
            [pltpu.SemaphoreType.DMA] * NB,
            [pltpu.SemaphoreType.DMA] * NB,
            [pltpu.SemaphoreType.DMA] * NB,
        ],
    )
    def emb(t0, t1, xtf, out, idxv, l0, l1, msk, rows01, tbuf,
            semi, semg0, semg1, semw):
        wid = lax.axis_index("s") * NC + lax.axis_index("c")
        c0 = wid * CPW
        iota = lax.broadcasted_iota(jnp.int32, (LANES,), 0)

        def xoff(c):
            return (c // (B // K)) * B + (c % (B // K)) * K

        def obase(c):
            l = c // (B // K)
            vb = c % (B // K)
            return (l * 8 * (B // K) + vb) * (8 * K)

        def owrite(pb, c, fire):
            ob = obase(c)
            for dt in range(DT):
                src = tbuf[pb].at[pl.ds(dt * 8 * K, 8 * K)]
                dst = out.at[pl.ds(ob + dt * (B // K) * 8 * K, 8 * K)]
                if fire:
                    pltpu.async_copy(src, dst, semw[pb])
                else:
                    pltpu.make_async_copy(src, dst, semw[pb]).wait()

        def process(pb, c):
            pltpu.make_async_copy(
                t0.at[l0[pb]], rows01[pb].at[pl.ds(0, K)], semg0[pb]).wait()
            pltpu.make_async_copy(
                t1.at[l1[pb]], rows01[pb].at[pl.ds(K, K)], semg1[pb]).wait()

            @pl.when(c - c0 >= NB)
            def _():
                owrite(pb, c - NB, fire=False)

            def col(g, _):
                blv = g * LANES + iota
                rowv = blv + msk[pb][pl.ds(g * LANES, LANES)] * K
                for jj in range(LANES):
                    rot = (iota + jj) & (LANES - 1)
                    rofs = rot * K + blv
                    for q in range(D // LANES):
                        dsrc = rot + q * LANES
                        val = plsc.load_gather(rows01[pb], [rowv, dsrc])
                        plsc.store_scatter(
                            tbuf[pb], [rofs + q * LANES * K], val)
                return 0

            lax.fori_loop(0, K // LANES, col, 0)
            owrite(pb, c, fire=True)

        for b in range(NB):
            pltpu.async_copy(
                xtf.at[pl.ds(xoff(c0 + b), K)], idxv[b], semi[b])

        def outer(jo, _):
            for b in range(NB):
                j = jo * NB + b
                c = c0 + j
                pltpu.make_async_copy(
                    xtf.at[pl.ds(xoff(c), K)], idxv[b], semi[b]).wait()

                def vec(v, _, _b=b):
                    sl = pl.ds(v * LANES, LANES)
                    vi = idxv[_b][sl]
                    m1 = vi >= V0
                    l0[_b][sl] = jnp.minimum(
                        jnp.where(m1, vi - V0, vi), V0 - 1)
                    l1[_b][sl] = jnp.where(
                        m1, jnp.minimum(vi - V0, V1 - 1), vi & 0xFFFF)
                    msk[_b][sl] = m1.astype(jnp.int32)
                    return 0

                lax.fori_loop(0, K // LANES, vec, 0)

                @pl.when(jo < NJO - 1)
                def _():
                    pltpu.async_copy(
                        xtf.at[pl.ds(xoff(c + NB), K)], idxv[b], semi[b])

                pltpu.async_copy(
                    t0.at[l0[b]], rows01[b].at[pl.ds(0, K)], semg0[b])
                pltpu.async_copy(
                    t1.at[l1[b]], rows01[b].at[pl.ds(K, K)], semg1[b])

                @pl.when(j > 0)
                def _(_pb=(b - 1) % NB, _c=c - 1):
                    process(_pb, _c)
            return 0

        lax.fori_loop(0, NJO, outer, 0)

        process(NB - 1, c0 + CPW - 1)
        for b in range(NB):
            owrite(b, c0 + CPW - NB + b, fire=False)

    return emb


@jax.jit
def kernel(table0, table1, x):
    xtf = x.T.reshape(B * L)
    flat = _build()(table0, table1, xtf)
    out5 = flat.reshape(L, D // 8, B // K, 8, K)
    return out5.transpose(2, 4, 0, 1, 3).reshape(B, L, D)
